# Initial kernel scaffold; baseline (speedup 1.0000x reference)
import jax
import jax.numpy as jnp
from jax.experimental import pallas as pl


def kernel(weights, starts, ends):
    raise NotImplementedError("write your pallas kernel here")


# speedup vs baseline: 2.7756x; 2.7756x over previous
import jax
import jax.numpy as jnp
from jax.experimental import pallas as pl
from jax.experimental.pallas import tpu as pltpu

FAR_PLANE = 10.0


def _body(w_ref, s_ref, e_ref, o_ref):
    w = w_ref[...]
    br, s = w.shape
    tri = (jax.lax.broadcasted_iota(jnp.int32, (s, s), 0)
           <= jax.lax.broadcasted_iota(jnp.int32, (s, s), 1)).astype(jnp.float32)
    cum = jax.lax.dot_general(w, tri, (((1,), (0,)), ((), ())),
                              preferred_element_type=jnp.float32,
                              precision=jax.lax.Precision.HIGHEST)
    idx = jnp.sum((cum < 0.5).astype(jnp.int32), axis=1, keepdims=True)
    steps = (s_ref[...] + e_ref[...]) * 0.5
    cols = jax.lax.broadcasted_iota(jnp.int32, (br, s), 1)
    onehot = (cols == idx).astype(jnp.float32)
    depth = jnp.sum(steps * onehot, axis=1, keepdims=True)
    depth = jnp.where(idx >= s, FAR_PLANE, depth)
    o_ref[...] = depth


def kernel(weights, starts, ends):
    b, s, _ = weights.shape
    w2 = weights[..., 0]
    s2 = starts[..., 0]
    e2 = ends[..., 0]
    br = 512
    grid = (b // br,)
    out = pl.pallas_call(
        _body,
        grid=grid,
        in_specs=[
            pl.BlockSpec((br, s), lambda i: (i, 0)),
            pl.BlockSpec((br, s), lambda i: (i, 0)),
            pl.BlockSpec((br, s), lambda i: (i, 0)),
        ],
        out_specs=pl.BlockSpec((br, 1), lambda i: (i, 0)),
        out_shape=jax.ShapeDtypeStruct((b, 1), jnp.float32),
    )(w2, s2, e2)
    return out


# speedup vs baseline: 8.4362x; 3.0394x over previous
import functools

import jax
import jax.numpy as jnp
from jax import lax
from jax.experimental import pallas as pl
from jax.experimental.pallas import tpu as pltpu
from jax.experimental.pallas import tpu_sc as plsc

FAR_PLANE = 10.0
B = 16384
S = 256
NW = 32
RPW = B // NW
NG = RPW // 16


def _sc_body(w_hbm, s_hbm, e_hbm, o_hbm,
             w16, w240, idxb, cntb, svb, evb, outb, sem):
    cid = lax.axis_index("c")
    sid = lax.axis_index("s")
    wid = sid * 2 + cid
    base = wid * RPW

    pltpu.sync_copy(w_hbm.at[pl.ds(base, RPW), pl.ds(0, 16)], w16)

    lanes = lax.iota(jnp.int32, 16)
    half = jnp.full((16,), 0.5, jnp.float32)
    one = jnp.full((16,), 1, jnp.int32)
    zero_i = jnp.full((16,), 0, jnp.int32)

    def group_body(g, _):
        row0 = g * 16
        rows = row0 + lanes

        def step(s, carry):
            acc, cnt = carry
            w = plsc.load_gather(w16, [rows, jnp.full((16,), s, jnp.int32)])
            acc = acc + w
            cnt = cnt + jnp.where(acc < half, one, zero_i)
            return acc, cnt

        acc0 = jnp.full((16,), 0.0, jnp.float32)
        acc, cnt = lax.fori_loop(0, 16, step, (acc0, zero_i))

        j = g // 8
        c0 = (g % 8) * 16
        cntb[j, pl.ds(c0, 16)] = cnt

        @pl.when(jnp.max(cnt) >= 16)
        def _slow():
            pltpu.sync_copy(
                w_hbm.at[pl.ds(base + row0, 16), pl.ds(16, S - 16)], w240)

            def step2(s, carry):
                acc2, cnt2 = carry
                w = plsc.load_gather(
                    w240, [lanes, jnp.full((16,), s, jnp.int32)])
                acc2 = acc2 + w
                cnt2 = cnt2 + jnp.where(acc2 < half, one, zero_i)
                return acc2, cnt2

            _, cnt2 = lax.fori_loop(0, S - 16, step2, (acc, cnt))
            cntb[j, pl.ds(c0, 16)] = cnt2

        cnt_f = cntb[j, pl.ds(c0, 16)]
        fidx = (base + row0 + lanes) * S + jnp.minimum(cnt_f, S - 1)
        idxb[j, pl.ds(c0, 16)] = fidx
        return 0

    lax.fori_loop(0, NG, group_body, 0)

    for j in range(4):
        pltpu.async_copy(s_hbm.at[idxb.at[j]], svb.at[j], sem).wait()
        pltpu.async_copy(e_hbm.at[idxb.at[j]], evb.at[j], sem).wait()

    far = jnp.full((16,), FAR_PLANE, jnp.float32)

    def out_body(t, _):
        j = t // 8
        c0 = (t % 8) * 16
        sv = svb[j, pl.ds(c0, 16)]
        ev = evb[j, pl.ds(c0, 16)]
        cf = cntb[j, pl.ds(c0, 16)]
        d = (sv + ev) * 0.5
        d = jnp.where(cf >= S, far, d)
        outb[pl.ds(t * 16, 16)] = d
        return 0

    lax.fori_loop(0, NG, out_body, 0)

    pltpu.sync_copy(outb, o_hbm.at[pl.ds(base, RPW)])


def kernel(weights, starts, ends):
    w2 = weights.reshape(B, S)
    sf = starts.reshape(B * S)
    ef = ends.reshape(B * S)
    mesh = plsc.VectorSubcoreMesh(core_axis_name="c", subcore_axis_name="s")
    k = functools.partial(
        pl.kernel,
        mesh=mesh,
        compiler_params=pltpu.CompilerParams(
            use_tc_tiling_on_sc=False, needs_layout_passes=False),
        out_type=jax.ShapeDtypeStruct((B,), jnp.float32),
        scratch_types=[
            pltpu.VMEM((RPW, 16), jnp.float32),
            pltpu.VMEM((16, S - 16), jnp.float32),
            pltpu.VMEM((4, 128), jnp.int32),
            pltpu.VMEM((4, 128), jnp.int32),
            pltpu.VMEM((4, 128), jnp.float32),
            pltpu.VMEM((4, 128), jnp.float32),
            pltpu.VMEM((RPW,), jnp.float32),
            pltpu.SemaphoreType.DMA,
        ],
    )(_sc_body)
    out = k(w2, sf, ef)
    return out.reshape(B, 1)


# speedup vs baseline: 10.5842x; 1.2546x over previous
import functools

import jax
import jax.numpy as jnp
from jax import lax
from jax.experimental import pallas as pl
from jax.experimental.pallas import tpu as pltpu
from jax.experimental.pallas import tpu_sc as plsc

FAR_PLANE = 10.0
B = 16384
S = 256
NW = 32
RPW = B // NW
NG = RPW // 16


def _sc_body(w_hbm, s_hbm, e_hbm, o_hbm,
             w16, w240, idxb, cntb, svb, evb, outb, sem):
    cid = lax.axis_index("c")
    sid = lax.axis_index("s")
    wid = sid * 2 + cid
    base = wid * RPW

    pltpu.sync_copy(w_hbm.at[pl.ds(base, RPW), pl.ds(0, 16)], w16)

    lanes = lax.iota(jnp.int32, 16)
    half = jnp.full((16,), 0.5, jnp.float32)
    one = jnp.full((16,), 1, jnp.int32)
    zero_i = jnp.full((16,), 0, jnp.int32)

    def group_body(g, _):
        row0 = g * 16
        rows = row0 + lanes

        acc = plsc.load_gather(w16, [rows, jnp.full((16,), 0, jnp.int32)])
        cnt = jnp.where(acc < half, one, zero_i)
        for s in range(1, 16):
            w = plsc.load_gather(w16, [rows, jnp.full((16,), s, jnp.int32)])
            acc = acc + w
            cnt = cnt + jnp.where(acc < half, one, zero_i)

        j = g // 8
        c0 = (g % 8) * 16
        cntb[j, pl.ds(c0, 16)] = cnt

        @pl.when(jnp.max(cnt) >= 16)
        def _slow():
            pltpu.sync_copy(
                w_hbm.at[pl.ds(base + row0, 16), pl.ds(16, S - 16)], w240)

            def step2(s, carry):
                acc2, cnt2 = carry
                w = plsc.load_gather(
                    w240, [lanes, jnp.full((16,), s, jnp.int32)])
                acc2 = acc2 + w
                cnt2 = cnt2 + jnp.where(acc2 < half, one, zero_i)
                return acc2, cnt2

            _, cnt2 = lax.fori_loop(0, S - 16, step2, (acc, cnt))
            cntb[j, pl.ds(c0, 16)] = cnt2

        cnt_f = cntb[j, pl.ds(c0, 16)]
        fidx = (base + row0 + lanes) * S + jnp.minimum(cnt_f, S - 1)
        idxb[j, pl.ds(c0, 16)] = fidx
        return 0

    lax.fori_loop(0, NG, group_body, 0)

    copies = []
    for j in range(4):
        copies.append(pltpu.async_copy(s_hbm.at[idxb.at[j]], svb.at[j], sem))
        copies.append(pltpu.async_copy(e_hbm.at[idxb.at[j]], evb.at[j], sem))
    for c in copies:
        c.wait()

    far = jnp.full((16,), FAR_PLANE, jnp.float32)

    def out_body(t, _):
        j = t // 8
        c0 = (t % 8) * 16
        sv = svb[j, pl.ds(c0, 16)]
        ev = evb[j, pl.ds(c0, 16)]
        cf = cntb[j, pl.ds(c0, 16)]
        d = (sv + ev) * 0.5
        d = jnp.where(cf >= S, far, d)
        outb[pl.ds(t * 16, 16)] = d
        return 0

    lax.fori_loop(0, NG, out_body, 0)

    pltpu.sync_copy(outb, o_hbm.at[pl.ds(base, RPW)])


def kernel(weights, starts, ends):
    w2 = weights.reshape(B, S)
    sf = starts.reshape(B * S)
    ef = ends.reshape(B * S)
    mesh = plsc.VectorSubcoreMesh(core_axis_name="c", subcore_axis_name="s")
    k = functools.partial(
        pl.kernel,
        mesh=mesh,
        compiler_params=pltpu.CompilerParams(
            use_tc_tiling_on_sc=False, needs_layout_passes=False),
        out_type=jax.ShapeDtypeStruct((B,), jnp.float32),
        scratch_types=[
            pltpu.VMEM((RPW, 16), jnp.float32),
            pltpu.VMEM((16, S - 16), jnp.float32),
            pltpu.VMEM((4, 128), jnp.int32),
            pltpu.VMEM((4, 128), jnp.int32),
            pltpu.VMEM((4, 128), jnp.float32),
            pltpu.VMEM((4, 128), jnp.float32),
            pltpu.VMEM((RPW,), jnp.float32),
            pltpu.SemaphoreType.DMA,
        ],
    )(_sc_body)
    out = k(w2, sf, ef)
    return out.reshape(B, 1)
